# baseline (speedup 1.0000x reference)
import functools

import jax
import jax.numpy as jnp
from jax import lax
from jax.experimental import pallas as pl
from jax.experimental.pallas import tpu as pltpu
from jax.experimental.pallas import tpu_sc as plsc

_NC = 2
_NS = 16
_NW = _NC * _NS

_B = 16384
_D = 64
_B_PER_W = _B // _NW
_CH = 16
_L = 16


_DGRP = 16


def _sc_gather_body(cid_hbm, did_hbm, ctabT_hbm, dtabT_hbm, outT_hbm,
                    idx_c, idx_d, xT_buf, sem):
  wid = lax.axis_index("s") * _NC + lax.axis_index("c")
  base = wid * _B_PER_W
  pltpu.sync_copy(cid_hbm.at[pl.ds(base, _B_PER_W)], idx_c)
  pltpu.sync_copy(did_hbm.at[pl.ds(base, _B_PER_W)], idx_d)

  for g in range(0, _D, _DGRP):
    copies = []
    for d in range(g, g + _DGRP):
      copies.append(pltpu.async_copy(
          ctabT_hbm.at[d].at[idx_c], xT_buf.at[d], sem))
      copies.append(pltpu.async_copy(
          dtabT_hbm.at[d].at[idx_d], xT_buf.at[_D + d], sem))
    for cp in copies:
      cp.wait()
  pltpu.sync_copy(xT_buf, outT_hbm.at[:, pl.ds(base, _B_PER_W)])


def _sc_gather(client_ids, cleaner_ids, ctabT, dtabT):
  mesh = plsc.VectorSubcoreMesh(core_axis_name="c", subcore_axis_name="s")
  fn = pl.kernel(
      _sc_gather_body,
      out_type=jax.ShapeDtypeStruct((2 * _D, _B), jnp.float32),
      mesh=mesh,
      scratch_types=[
          pltpu.VMEM((_B_PER_W,), jnp.int32),
          pltpu.VMEM((_B_PER_W,), jnp.int32),
          pltpu.VMEM((2 * _D, _B_PER_W), jnp.float32),
          pltpu.SemaphoreType.DMA,
      ],
      compiler_params=pltpu.CompilerParams(use_tc_tiling_on_sc=False),
  )
  return fn(client_ids, cleaner_ids, ctabT, dtabT)


_MLP_BLK = 4096


def _mlp_body(xT_ref, w1t_ref, b1_ref, w2t_ref, b2_ref,
              w3t_ref, b3_ref, w4t_ref, b4_ref, out_ref):
  h = jnp.maximum(w1t_ref[...] @ xT_ref[...] + b1_ref[...], 0.0)
  h = jnp.maximum(w2t_ref[...] @ h + b2_ref[...], 0.0)
  h = jnp.maximum(w3t_ref[...] @ h + b3_ref[...], 0.0)
  out_ref[...] = w4t_ref[...] @ h + b4_ref[...]


def _mlp(xT, W1, b1, W2, b2, W3, b3, W4, b4):
  grid = (_B // _MLP_BLK,)
  full = lambda shape: pl.BlockSpec(shape, lambda i: (0, 0))
  return pl.pallas_call(
      _mlp_body,
      grid=grid,
      in_specs=[
          pl.BlockSpec((2 * _D, _MLP_BLK), lambda i: (0, i)),
          full((128, 128)),
          full((128, 1)),
          full((64, 128)),
          full((64, 1)),
          full((32, 64)),
          full((32, 1)),
          full((1, 32)),
          full((1, 1)),
      ],
      out_specs=pl.BlockSpec((1, _MLP_BLK), lambda i: (0, i)),
      out_shape=jax.ShapeDtypeStruct((1, _B), jnp.float32),
  )(xT, W1.T, b1.reshape(-1, 1), W2.T, b2.reshape(-1, 1),
    W3.T, b3.reshape(-1, 1), W4.T, b4.reshape(1, 1))


@jax.jit
def kernel(client_ids, cleaner_ids, client_table, cleaner_table,
           W1, b1, W2, b2, W3, b3, W4, b4):
  xT = _sc_gather(client_ids.astype(jnp.int32),
                  cleaner_ids.astype(jnp.int32),
                  client_table.T, cleaner_table.T)
  out = _mlp(xT, W1, b1, W2, b2, W3, b3, W4, b4)
  return out.reshape(_B)


# speedup vs baseline: 12.0096x; 12.0096x over previous
import functools

import jax
import jax.numpy as jnp
from jax import lax
from jax.experimental import pallas as pl
from jax.experimental.pallas import tpu as pltpu
from jax.experimental.pallas import tpu_sc as plsc

_NC = 2
_NS = 16
_NW = _NC * _NS

_B = 16384
_D = 64
_N = 1000000
_B_PER_W = _B // _NW
_L = 16


_RP_CH = 8192


def _repack_body(tabT_ref, out_ref):
  a = tabT_ref[...]
  at = a.T
  at3 = at.reshape(_RP_CH // 2, 2, _D)
  out_ref[...] = jnp.concatenate([at3[:, 0, :], at3[:, 1, :]], axis=-1)


def _repack(tabT):
  grid = ((_N + _RP_CH - 1) // _RP_CH,)
  return pl.pallas_call(
      _repack_body,
      grid=grid,
      in_specs=[pl.BlockSpec((_D, _RP_CH), lambda i: (0, i))],
      out_specs=pl.BlockSpec((_RP_CH // 2, 2 * _D), lambda i: (i, 0)),
      out_shape=jax.ShapeDtypeStruct((_N // 2, 2 * _D), jnp.float32),
  )(tabT)


def _sc_gather_body(cid_hbm, did_hbm, yc_hbm, yd_hbm, out_c_hbm, out_d_hbm,
                    idx_c, idx_d, tid, ybuf, sem):
  wid = lax.axis_index("s") * _NC + lax.axis_index("c")
  base = wid * _B_PER_W
  pltpu.sync_copy(cid_hbm.at[pl.ds(base, _B_PER_W)], idx_c)
  pltpu.sync_copy(did_hbm.at[pl.ds(base, _B_PER_W)], idx_d)

  def halve_c(i, _):
    tid[pl.ds(i * _L, _L)] = lax.shift_right_logical(idx_c[pl.ds(i * _L, _L)], 1)
    return 0

  def halve_d(i, _):
    tid[pl.ds(i * _L, _L)] = lax.shift_right_logical(idx_d[pl.ds(i * _L, _L)], 1)
    return 0

  lax.fori_loop(0, _B_PER_W // _L, halve_c, 0)
  pltpu.async_copy(yc_hbm.at[tid], ybuf, sem).wait()
  pltpu.sync_copy(ybuf, out_c_hbm.at[pl.ds(base, _B_PER_W)])

  lax.fori_loop(0, _B_PER_W // _L, halve_d, 0)
  pltpu.async_copy(yd_hbm.at[tid], ybuf, sem).wait()
  pltpu.sync_copy(ybuf, out_d_hbm.at[pl.ds(base, _B_PER_W)])


def _sc_gather(client_ids, cleaner_ids, yc, yd):
  mesh = plsc.VectorSubcoreMesh(core_axis_name="c", subcore_axis_name="s")
  fn = pl.kernel(
      _sc_gather_body,
      out_type=[
          jax.ShapeDtypeStruct((_B, 2 * _D), jnp.float32),
          jax.ShapeDtypeStruct((_B, 2 * _D), jnp.float32),
      ],
      mesh=mesh,
      scratch_types=[
          pltpu.VMEM((_B_PER_W,), jnp.int32),
          pltpu.VMEM((_B_PER_W,), jnp.int32),
          pltpu.VMEM((_B_PER_W,), jnp.int32),
          pltpu.VMEM((_B_PER_W, 2 * _D), jnp.float32),
          pltpu.SemaphoreType.DMA,
      ],
  )
  return fn(client_ids, cleaner_ids, yc, yd)


_MLP_BLK = 2048
_NBLK = _B // _MLP_BLK


def _mlp_body(yc_ref, yd_ref, cid_ref, did_ref, w1a_ref, w1b_ref, b1_ref,
              w2_ref, b2_ref, w3_ref, b3_ref, w4_ref, b4_ref, out_ref):
  cbit = (cid_ref[0, 0, :] & 1).reshape(_MLP_BLK, 1)
  dbit = (did_ref[0, 0, :] & 1).reshape(_MLP_BLK, 1)
  yc = yc_ref[...]
  yd = yd_ref[...]
  xc = jnp.where(cbit == 0, yc[:, :_D], yc[:, _D:])
  xd = jnp.where(dbit == 0, yd[:, :_D], yd[:, _D:])
  h = jnp.maximum(xc @ w1a_ref[...] + xd @ w1b_ref[...] + b1_ref[...], 0.0)
  h = jnp.maximum(h @ w2_ref[...] + b2_ref[...], 0.0)
  h = jnp.maximum(h @ w3_ref[...] + b3_ref[...], 0.0)
  out_ref[...] = h @ w4_ref[...] + b4_ref[...]


def _mlp(yc, yd, cid3, did3, W1, b1, W2, b2, W3, b3, W4, b4):
  grid = (_NBLK,)
  full = lambda shape: pl.BlockSpec(shape, lambda i: tuple(0 for _ in shape))
  return pl.pallas_call(
      _mlp_body,
      grid=grid,
      in_specs=[
          pl.BlockSpec((_MLP_BLK, 2 * _D), lambda i: (i, 0)),
          pl.BlockSpec((_MLP_BLK, 2 * _D), lambda i: (i, 0)),
          pl.BlockSpec((1, 1, _MLP_BLK), lambda i: (i, 0, 0)),
          pl.BlockSpec((1, 1, _MLP_BLK), lambda i: (i, 0, 0)),
          full((_D, 128)),
          full((_D, 128)),
          full((1, 128)),
          full((128, 64)),
          full((1, 64)),
          full((64, 32)),
          full((1, 32)),
          full((32, 1)),
          full((1, 1)),
      ],
      out_specs=pl.BlockSpec((_MLP_BLK, 1), lambda i: (i, 0)),
      out_shape=jax.ShapeDtypeStruct((_B, 1), jnp.float32),
  )(yc, yd, cid3, did3, W1[:_D], W1[_D:], b1.reshape(1, -1),
    W2, b2.reshape(1, -1), W3, b3.reshape(1, -1), W4, b4.reshape(1, 1))


@jax.jit
def kernel(client_ids, cleaner_ids, client_table, cleaner_table,
           W1, b1, W2, b2, W3, b3, W4, b4):
  cid = client_ids.astype(jnp.int32)
  did = cleaner_ids.astype(jnp.int32)
  yc_tab = _repack(client_table.T)
  yd_tab = _repack(cleaner_table.T)
  yc, yd = _sc_gather(cid, did, yc_tab, yd_tab)
  cid3 = cid.reshape(_NBLK, 1, _MLP_BLK)
  did3 = did.reshape(_NBLK, 1, _MLP_BLK)
  out = _mlp(yc, yd, cid3, did3, W1, b1, W2, b2, W3, b3, W4, b4)
  return out.reshape(_B)


# speedup vs baseline: 18.0629x; 1.5040x over previous
import functools

import jax
import jax.numpy as jnp
from jax import lax
from jax.experimental import pallas as pl
from jax.experimental.pallas import tpu as pltpu
from jax.experimental.pallas import tpu_sc as plsc

_NC = 2
_NS = 16
_NW = _NC * _NS

_B = 16384
_D = 64
_N = 1000000
_B_PER_W = _B // _NW
_L = 16


_RP_CH = 4096
_NSTEP = 123
_HALF = _RP_CH * _NSTEP


def _repack_body(lo_ref, hi_ref, out_ref):
  out_ref[...] = jnp.concatenate([lo_ref[...].T, hi_ref[...].T], axis=-1)


def _repack(tabT):
  grid = (_NSTEP,)
  return pl.pallas_call(
      _repack_body,
      grid=grid,
      in_specs=[
          pl.BlockSpec((_D, _RP_CH), lambda i: (0, i)),
          pl.BlockSpec(
              (_D, _RP_CH),
              lambda i: (0, jnp.minimum(i + _NSTEP, _N // _RP_CH)),
          ),
      ],
      out_specs=pl.BlockSpec((_RP_CH, 2 * _D), lambda i: (i, 0)),
      out_shape=jax.ShapeDtypeStruct((_HALF, 2 * _D), jnp.float32),
  )(tabT, tabT)


def _sc_gather_body(cid_hbm, did_hbm, yc_hbm, yd_hbm, out_c_hbm, out_d_hbm,
                    idx_c, idx_d, tid, ybuf, sem):
  wid = lax.axis_index("s") * _NC + lax.axis_index("c")
  base = wid * _B_PER_W
  pltpu.sync_copy(cid_hbm.at[pl.ds(base, _B_PER_W)], idx_c)
  pltpu.sync_copy(did_hbm.at[pl.ds(base, _B_PER_W)], idx_d)

  def halve_c(i, _):
    v = idx_c[pl.ds(i * _L, _L)]
    tid[pl.ds(i * _L, _L)] = jnp.where(v < _HALF, v, v - _HALF)
    return 0

  def halve_d(i, _):
    v = idx_d[pl.ds(i * _L, _L)]
    tid[pl.ds(i * _L, _L)] = jnp.where(v < _HALF, v, v - _HALF)
    return 0

  lax.fori_loop(0, _B_PER_W // _L, halve_c, 0)
  pltpu.async_copy(yc_hbm.at[tid], ybuf, sem).wait()
  pltpu.sync_copy(ybuf, out_c_hbm.at[pl.ds(base, _B_PER_W)])

  lax.fori_loop(0, _B_PER_W // _L, halve_d, 0)
  pltpu.async_copy(yd_hbm.at[tid], ybuf, sem).wait()
  pltpu.sync_copy(ybuf, out_d_hbm.at[pl.ds(base, _B_PER_W)])


def _sc_gather(client_ids, cleaner_ids, yc, yd):
  mesh = plsc.VectorSubcoreMesh(core_axis_name="c", subcore_axis_name="s")
  fn = pl.kernel(
      _sc_gather_body,
      out_type=[
          jax.ShapeDtypeStruct((_B, 2 * _D), jnp.float32),
          jax.ShapeDtypeStruct((_B, 2 * _D), jnp.float32),
      ],
      mesh=mesh,
      scratch_types=[
          pltpu.VMEM((_B_PER_W,), jnp.int32),
          pltpu.VMEM((_B_PER_W,), jnp.int32),
          pltpu.VMEM((_B_PER_W,), jnp.int32),
          pltpu.VMEM((_B_PER_W, 2 * _D), jnp.float32),
          pltpu.SemaphoreType.DMA,
      ],
  )
  return fn(client_ids, cleaner_ids, yc, yd)


_MLP_BLK = 2048
_NBLK = _B // _MLP_BLK


def _mlp_body(yc_ref, yd_ref, cid_ref, did_ref, w1a_ref, w1b_ref, b1_ref,
              w2_ref, b2_ref, w3_ref, b3_ref, w4_ref, b4_ref, out_ref):
  cbit = cid_ref[0, 0, :].reshape(_MLP_BLK, 1)
  dbit = did_ref[0, 0, :].reshape(_MLP_BLK, 1)
  yc = yc_ref[...]
  yd = yd_ref[...]
  xc = jnp.where(cbit < _HALF, yc[:, :_D], yc[:, _D:])
  xd = jnp.where(dbit < _HALF, yd[:, :_D], yd[:, _D:])
  h = jnp.maximum(xc @ w1a_ref[...] + xd @ w1b_ref[...] + b1_ref[...], 0.0)
  h = jnp.maximum(h @ w2_ref[...] + b2_ref[...], 0.0)
  h = jnp.maximum(h @ w3_ref[...] + b3_ref[...], 0.0)
  out_ref[...] = h @ w4_ref[...] + b4_ref[...]


def _mlp(yc, yd, cid3, did3, W1, b1, W2, b2, W3, b3, W4, b4):
  grid = (_NBLK,)
  full = lambda shape: pl.BlockSpec(shape, lambda i: tuple(0 for _ in shape))
  return pl.pallas_call(
      _mlp_body,
      grid=grid,
      in_specs=[
          pl.BlockSpec((_MLP_BLK, 2 * _D), lambda i: (i, 0)),
          pl.BlockSpec((_MLP_BLK, 2 * _D), lambda i: (i, 0)),
          pl.BlockSpec((1, 1, _MLP_BLK), lambda i: (i, 0, 0)),
          pl.BlockSpec((1, 1, _MLP_BLK), lambda i: (i, 0, 0)),
          full((_D, 128)),
          full((_D, 128)),
          full((1, 128)),
          full((128, 64)),
          full((1, 64)),
          full((64, 32)),
          full((1, 32)),
          full((32, 1)),
          full((1, 1)),
      ],
      out_specs=pl.BlockSpec((_MLP_BLK, 1), lambda i: (i, 0)),
      out_shape=jax.ShapeDtypeStruct((_B, 1), jnp.float32),
  )(yc, yd, cid3, did3, W1[:_D], W1[_D:], b1.reshape(1, -1),
    W2, b2.reshape(1, -1), W3, b3.reshape(1, -1), W4, b4.reshape(1, 1))


@jax.jit
def kernel(client_ids, cleaner_ids, client_table, cleaner_table,
           W1, b1, W2, b2, W3, b3, W4, b4):
  cid = client_ids.astype(jnp.int32)
  did = cleaner_ids.astype(jnp.int32)
  yc_tab = _repack(client_table.T)
  yd_tab = _repack(cleaner_table.T)
  yc, yd = _sc_gather(cid, did, yc_tab, yd_tab)
  cid3 = cid.reshape(_NBLK, 1, _MLP_BLK)
  did3 = did.reshape(_NBLK, 1, _MLP_BLK)
  out = _mlp(yc, yd, cid3, did3, W1, b1, W2, b2, W3, b3, W4, b4)
  return out.reshape(_B)


# speedup vs baseline: 27.3766x; 1.5156x over previous
import functools

import jax
import jax.numpy as jnp
from jax import lax
from jax.experimental import pallas as pl
from jax.experimental.pallas import tpu as pltpu
from jax.experimental.pallas import tpu_sc as plsc

_NC = 2
_NS = 16
_NW = _NC * _NS

_B = 16384
_D = 64
_N = 1000000
_B_PER_W = _B // _NW
_L = 16


_RP_CH = 4096
_NSTEP = 62
_HALF = _RP_CH * _NSTEP
_MAXBLK = _N // _RP_CH


def _repack_body(r0_ref, r1_ref, r2_ref, r3_ref, out_ref):
  ylo = jnp.concatenate([r0_ref[...].T, r2_ref[...].T], axis=-1)
  yhi = jnp.concatenate([r1_ref[...].T, r3_ref[...].T], axis=-1)
  blo = lax.bitcast_convert_type(
      ylo.astype(jnp.bfloat16).astype(jnp.float32), jnp.uint32)
  bhi = lax.bitcast_convert_type(
      yhi.astype(jnp.bfloat16).astype(jnp.float32), jnp.uint32)
  packed = jnp.right_shift(blo, jnp.uint32(16)) | bhi
  out_ref[...] = lax.bitcast_convert_type(packed, jnp.float32)


def _repack(tabT):
  grid = (_NSTEP,)

  def spec(k):
    return pl.BlockSpec(
        (_D, _RP_CH), lambda i: (0, jnp.minimum(i + k * _NSTEP, _MAXBLK)))

  return pl.pallas_call(
      _repack_body,
      grid=grid,
      in_specs=[spec(0), spec(1), spec(2), spec(3)],
      out_specs=pl.BlockSpec((_RP_CH, 2 * _D), lambda i: (i, 0)),
      out_shape=jax.ShapeDtypeStruct((_HALF, 2 * _D), jnp.float32),
  )(tabT, tabT, tabT, tabT)


def _sc_gather_body(cid_hbm, did_hbm, yc_hbm, yd_hbm, out_c_hbm, out_d_hbm,
                    idx_c, idx_d, tid, ybuf, sem):
  wid = lax.axis_index("s") * _NC + lax.axis_index("c")
  base = wid * _B_PER_W
  pltpu.sync_copy(cid_hbm.at[pl.ds(base, _B_PER_W)], idx_c)
  pltpu.sync_copy(did_hbm.at[pl.ds(base, _B_PER_W)], idx_d)

  def halve_c(i, _):
    v = idx_c[pl.ds(i * _L, _L)]
    v = jnp.where(v < 2 * _HALF, v, v - 2 * _HALF)
    tid[pl.ds(i * _L, _L)] = jnp.where(v < _HALF, v, v - _HALF)
    return 0

  def halve_d(i, _):
    v = idx_d[pl.ds(i * _L, _L)]
    v = jnp.where(v < 2 * _HALF, v, v - 2 * _HALF)
    tid[pl.ds(i * _L, _L)] = jnp.where(v < _HALF, v, v - _HALF)
    return 0

  lax.fori_loop(0, _B_PER_W // _L, halve_c, 0)
  pltpu.async_copy(yc_hbm.at[tid], ybuf, sem).wait()
  pltpu.sync_copy(ybuf, out_c_hbm.at[pl.ds(base, _B_PER_W)])

  lax.fori_loop(0, _B_PER_W // _L, halve_d, 0)
  pltpu.async_copy(yd_hbm.at[tid], ybuf, sem).wait()
  pltpu.sync_copy(ybuf, out_d_hbm.at[pl.ds(base, _B_PER_W)])


def _sc_gather(client_ids, cleaner_ids, yc, yd):
  mesh = plsc.VectorSubcoreMesh(core_axis_name="c", subcore_axis_name="s")
  fn = pl.kernel(
      _sc_gather_body,
      out_type=[
          jax.ShapeDtypeStruct((_B, 2 * _D), jnp.float32),
          jax.ShapeDtypeStruct((_B, 2 * _D), jnp.float32),
      ],
      mesh=mesh,
      scratch_types=[
          pltpu.VMEM((_B_PER_W,), jnp.int32),
          pltpu.VMEM((_B_PER_W,), jnp.int32),
          pltpu.VMEM((_B_PER_W,), jnp.int32),
          pltpu.VMEM((_B_PER_W, 2 * _D), jnp.float32),
          pltpu.SemaphoreType.DMA,
      ],
  )
  return fn(client_ids, cleaner_ids, yc, yd)


_MLP_BLK = 2048
_NBLK = _B // _MLP_BLK


def _mlp_body(yc_ref, yd_ref, cid_ref, did_ref, w1a_ref, w1b_ref, b1_ref,
              w2_ref, b2_ref, w3_ref, b3_ref, w4_ref, b4_ref, out_ref):
  cbit = cid_ref[0, 0, :].reshape(_MLP_BLK, 1)
  dbit = did_ref[0, 0, :].reshape(_MLP_BLK, 1)

  def quarter(y_ref, b):
    u = lax.bitcast_convert_type(y_ref[...], jnp.uint32)
    q = ((b >= _HALF).astype(jnp.int32) + (b >= 2 * _HALF).astype(jnp.int32)
         + (b >= 3 * _HALF).astype(jnp.int32))
    ge2 = q >= 2
    odd = (q & 1) == 1
    uhalf = jnp.where(ge2, u[:, _D:], u[:, :_D])
    ubits = jnp.where(odd, uhalf & jnp.uint32(0xFFFF0000),
                      jnp.left_shift(uhalf, jnp.uint32(16)))
    return lax.bitcast_convert_type(ubits, jnp.float32)

  xc = quarter(yc_ref, cbit)
  xd = quarter(yd_ref, dbit)
  h = jnp.maximum(xc @ w1a_ref[...] + xd @ w1b_ref[...] + b1_ref[...], 0.0)
  h = jnp.maximum(h @ w2_ref[...] + b2_ref[...], 0.0)
  h = jnp.maximum(h @ w3_ref[...] + b3_ref[...], 0.0)
  out_ref[...] = h @ w4_ref[...] + b4_ref[...]


def _mlp(yc, yd, cid3, did3, W1, b1, W2, b2, W3, b3, W4, b4):
  grid = (_NBLK,)
  full = lambda shape: pl.BlockSpec(shape, lambda i: tuple(0 for _ in shape))
  return pl.pallas_call(
      _mlp_body,
      grid=grid,
      in_specs=[
          pl.BlockSpec((_MLP_BLK, 2 * _D), lambda i: (i, 0)),
          pl.BlockSpec((_MLP_BLK, 2 * _D), lambda i: (i, 0)),
          pl.BlockSpec((1, 1, _MLP_BLK), lambda i: (i, 0, 0)),
          pl.BlockSpec((1, 1, _MLP_BLK), lambda i: (i, 0, 0)),
          full((_D, 128)),
          full((_D, 128)),
          full((1, 128)),
          full((128, 64)),
          full((1, 64)),
          full((64, 32)),
          full((1, 32)),
          full((32, 1)),
          full((1, 1)),
      ],
      out_specs=pl.BlockSpec((_MLP_BLK, 1), lambda i: (i, 0)),
      out_shape=jax.ShapeDtypeStruct((_B, 1), jnp.float32),
  )(yc, yd, cid3, did3, W1[:_D], W1[_D:], b1.reshape(1, -1),
    W2, b2.reshape(1, -1), W3, b3.reshape(1, -1), W4, b4.reshape(1, 1))


@jax.jit
def kernel(client_ids, cleaner_ids, client_table, cleaner_table,
           W1, b1, W2, b2, W3, b3, W4, b4):
  cid = client_ids.astype(jnp.int32)
  did = cleaner_ids.astype(jnp.int32)
  yc_tab = _repack(client_table.T)
  yd_tab = _repack(cleaner_table.T)
  yc, yd = _sc_gather(cid, did, yc_tab, yd_tab)
  cid3 = cid.reshape(_NBLK, 1, _MLP_BLK)
  did3 = did.reshape(_NBLK, 1, _MLP_BLK)
  out = _mlp(yc, yd, cid3, did3, W1, b1, W2, b2, W3, b3, W4, b4)
  return out.reshape(_B)


# speedup vs baseline: 28.4303x; 1.0385x over previous
import functools

import jax
import jax.numpy as jnp
from jax import lax
from jax.experimental import pallas as pl
from jax.experimental.pallas import tpu as pltpu
from jax.experimental.pallas import tpu_sc as plsc

_NC = 2
_NS = 16
_NW = _NC * _NS

_B = 16384
_D = 64
_N = 1000000
_B_PER_W = _B // _NW
_L = 16


_RP_CH = 4096
_NSTEP = 62
_HALF = _RP_CH * _NSTEP
_MAXBLK = _N // _RP_CH


def _bf16_bits(x):
  return lax.bitcast_convert_type(
      x.astype(jnp.bfloat16).astype(jnp.float32), jnp.uint32)


def _repack_body(r0_ref, r1_ref, r2_ref, r3_ref, out_ref):
  z01 = (jnp.right_shift(_bf16_bits(r0_ref[...]), jnp.uint32(16))
         | (_bf16_bits(r1_ref[...]) & jnp.uint32(0xFFFF0000)))
  z23 = (jnp.right_shift(_bf16_bits(r2_ref[...]), jnp.uint32(16))
         | (_bf16_bits(r3_ref[...]) & jnp.uint32(0xFFFF0000)))
  packed = jnp.concatenate([z01.T, z23.T], axis=-1)
  out_ref[...] = lax.bitcast_convert_type(packed, jnp.float32)


def _repack(tabT):
  grid = (_NSTEP,)

  def spec(k):
    return pl.BlockSpec(
        (_D, _RP_CH), lambda i: (0, jnp.minimum(i + k * _NSTEP, _MAXBLK)))

  return pl.pallas_call(
      _repack_body,
      grid=grid,
      in_specs=[spec(0), spec(1), spec(2), spec(3)],
      out_specs=pl.BlockSpec((_RP_CH, 2 * _D), lambda i: (i, 0)),
      out_shape=jax.ShapeDtypeStruct((_HALF, 2 * _D), jnp.float32),
  )(tabT, tabT, tabT, tabT)


def _sc_gather_body(cid_hbm, did_hbm, yc_hbm, yd_hbm, out_c_hbm, out_d_hbm,
                    idx_c, idx_d, tid, ybuf, sem):
  wid = lax.axis_index("s") * _NC + lax.axis_index("c")
  base = wid * _B_PER_W
  pltpu.sync_copy(cid_hbm.at[pl.ds(base, _B_PER_W)], idx_c)
  pltpu.sync_copy(did_hbm.at[pl.ds(base, _B_PER_W)], idx_d)

  def halve_c(i, _):
    v = idx_c[pl.ds(i * _L, _L)]
    v = jnp.where(v < 2 * _HALF, v, v - 2 * _HALF)
    tid[pl.ds(i * _L, _L)] = jnp.where(v < _HALF, v, v - _HALF)
    return 0

  def halve_d(i, _):
    v = idx_d[pl.ds(i * _L, _L)]
    v = jnp.where(v < 2 * _HALF, v, v - 2 * _HALF)
    tid[pl.ds(i * _L, _L)] = jnp.where(v < _HALF, v, v - _HALF)
    return 0

  lax.fori_loop(0, _B_PER_W // _L, halve_c, 0)
  pltpu.async_copy(yc_hbm.at[tid], ybuf, sem).wait()
  pltpu.sync_copy(ybuf, out_c_hbm.at[pl.ds(base, _B_PER_W)])

  lax.fori_loop(0, _B_PER_W // _L, halve_d, 0)
  pltpu.async_copy(yd_hbm.at[tid], ybuf, sem).wait()
  pltpu.sync_copy(ybuf, out_d_hbm.at[pl.ds(base, _B_PER_W)])


def _sc_gather(client_ids, cleaner_ids, yc, yd):
  mesh = plsc.VectorSubcoreMesh(core_axis_name="c", subcore_axis_name="s")
  fn = pl.kernel(
      _sc_gather_body,
      out_type=[
          jax.ShapeDtypeStruct((_B, 2 * _D), jnp.float32),
          jax.ShapeDtypeStruct((_B, 2 * _D), jnp.float32),
      ],
      mesh=mesh,
      scratch_types=[
          pltpu.VMEM((_B_PER_W,), jnp.int32),
          pltpu.VMEM((_B_PER_W,), jnp.int32),
          pltpu.VMEM((_B_PER_W,), jnp.int32),
          pltpu.VMEM((_B_PER_W, 2 * _D), jnp.float32),
          pltpu.SemaphoreType.DMA,
      ],
  )
  return fn(client_ids, cleaner_ids, yc, yd)


_MLP_BLK = 2048
_NBLK = _B // _MLP_BLK


def _mlp_body(yc_ref, yd_ref, cid_ref, did_ref, w1a_ref, w1b_ref, b1_ref,
              w2_ref, b2_ref, w3_ref, b3_ref, w4_ref, b4_ref, out_ref):
  cbit = cid_ref[0, 0, :].reshape(_MLP_BLK, 1)
  dbit = did_ref[0, 0, :].reshape(_MLP_BLK, 1)

  def quarter(y_ref, b):
    u = lax.bitcast_convert_type(y_ref[...], jnp.uint32)
    q = ((b >= _HALF).astype(jnp.int32) + (b >= 2 * _HALF).astype(jnp.int32)
         + (b >= 3 * _HALF).astype(jnp.int32))
    ge2 = q >= 2
    odd = (q & 1) == 1
    uhalf = jnp.where(ge2, u[:, _D:], u[:, :_D])
    ubits = jnp.where(odd, uhalf & jnp.uint32(0xFFFF0000),
                      jnp.left_shift(uhalf, jnp.uint32(16)))
    return lax.bitcast_convert_type(ubits, jnp.float32)

  xc = quarter(yc_ref, cbit)
  xd = quarter(yd_ref, dbit)
  h = jnp.maximum(xc @ w1a_ref[...] + xd @ w1b_ref[...] + b1_ref[...], 0.0)
  h = jnp.maximum(h @ w2_ref[...] + b2_ref[...], 0.0)
  h = jnp.maximum(h @ w3_ref[...] + b3_ref[...], 0.0)
  out_ref[...] = h @ w4_ref[...] + b4_ref[...]


def _mlp(yc, yd, cid3, did3, W1, b1, W2, b2, W3, b3, W4, b4):
  grid = (_NBLK,)
  full = lambda shape: pl.BlockSpec(shape, lambda i: tuple(0 for _ in shape))
  return pl.pallas_call(
      _mlp_body,
      grid=grid,
      in_specs=[
          pl.BlockSpec((_MLP_BLK, 2 * _D), lambda i: (i, 0)),
          pl.BlockSpec((_MLP_BLK, 2 * _D), lambda i: (i, 0)),
          pl.BlockSpec((1, 1, _MLP_BLK), lambda i: (i, 0, 0)),
          pl.BlockSpec((1, 1, _MLP_BLK), lambda i: (i, 0, 0)),
          full((_D, 128)),
          full((_D, 128)),
          full((1, 128)),
          full((128, 64)),
          full((1, 64)),
          full((64, 32)),
          full((1, 32)),
          full((32, 1)),
          full((1, 1)),
      ],
      out_specs=pl.BlockSpec((_MLP_BLK, 1), lambda i: (i, 0)),
      out_shape=jax.ShapeDtypeStruct((_B, 1), jnp.float32),
  )(yc, yd, cid3, did3, W1[:_D], W1[_D:], b1.reshape(1, -1),
    W2, b2.reshape(1, -1), W3, b3.reshape(1, -1), W4, b4.reshape(1, 1))


@jax.jit
def kernel(client_ids, cleaner_ids, client_table, cleaner_table,
           W1, b1, W2, b2, W3, b3, W4, b4):
  cid = client_ids.astype(jnp.int32)
  did = cleaner_ids.astype(jnp.int32)
  yc_tab = _repack(client_table.T)
  yd_tab = _repack(cleaner_table.T)
  yc, yd = _sc_gather(cid, did, yc_tab, yd_tab)
  cid3 = cid.reshape(_NBLK, 1, _MLP_BLK)
  did3 = did.reshape(_NBLK, 1, _MLP_BLK)
  out = _mlp(yc, yd, cid3, did3, W1, b1, W2, b2, W3, b3, W4, b4)
  return out.reshape(_B)


# speedup vs baseline: 28.7892x; 1.0126x over previous
import functools

import jax
import jax.numpy as jnp
from jax import lax
from jax.experimental import pallas as pl
from jax.experimental.pallas import tpu as pltpu
from jax.experimental.pallas import tpu_sc as plsc

_NC = 2
_NS = 16
_NW = _NC * _NS

_B = 16384
_D = 64
_N = 1000000
_B_PER_W = _B // _NW
_L = 16


_RP_CH = 4096
_NSTEP = 62
_HALF = _RP_CH * _NSTEP
_MAXBLK = _N // _RP_CH


def _bf16_bits(x):
  return lax.bitcast_convert_type(
      x.astype(jnp.bfloat16).astype(jnp.float32), jnp.uint32)


def _repack_body(r0_ref, r1_ref, r2_ref, r3_ref, out_ref):
  z01 = (jnp.right_shift(_bf16_bits(r0_ref[...]), jnp.uint32(16))
         | (_bf16_bits(r1_ref[...]) & jnp.uint32(0xFFFF0000)))
  z23 = (jnp.right_shift(_bf16_bits(r2_ref[...]), jnp.uint32(16))
         | (_bf16_bits(r3_ref[...]) & jnp.uint32(0xFFFF0000)))
  packed = jnp.concatenate([z01.T, z23.T], axis=-1)
  out_ref[...] = lax.bitcast_convert_type(packed, jnp.float32)


def _repack(tabT):
  grid = (_NSTEP,)

  def spec(k):
    return pl.BlockSpec(
        (_D, _RP_CH), lambda i: (0, jnp.minimum(i + k * _NSTEP, _MAXBLK)))

  return pl.pallas_call(
      _repack_body,
      grid=grid,
      in_specs=[spec(0), spec(1), spec(2), spec(3)],
      out_specs=pl.BlockSpec((_RP_CH, 2 * _D), lambda i: (i, 0)),
      out_shape=jax.ShapeDtypeStruct((_HALF, 2 * _D), jnp.float32),
  )(tabT, tabT, tabT, tabT)


def _sc_gather_body(ids_hbm, y_hbm, out_hbm, idx, tid, ybuf, sem):
  wid = lax.axis_index("s") * _NC + lax.axis_index("c")
  base = wid * _B_PER_W
  pltpu.sync_copy(ids_hbm.at[pl.ds(base, _B_PER_W)], idx)

  def fold(i, _):
    v = idx[pl.ds(i * _L, _L)]
    v = jnp.where(v < 2 * _HALF, v, v - 2 * _HALF)
    tid[pl.ds(i * _L, _L)] = jnp.where(v < _HALF, v, v - _HALF)
    return 0

  lax.fori_loop(0, _B_PER_W // _L, fold, 0)
  pltpu.async_copy(y_hbm.at[tid], ybuf, sem).wait()
  pltpu.sync_copy(ybuf, out_hbm.at[pl.ds(base, _B_PER_W)])


def _sc_gather(ids, y):
  mesh = plsc.VectorSubcoreMesh(core_axis_name="c", subcore_axis_name="s")
  fn = pl.kernel(
      _sc_gather_body,
      out_type=jax.ShapeDtypeStruct((_B, 2 * _D), jnp.float32),
      mesh=mesh,
      scratch_types=[
          pltpu.VMEM((_B_PER_W,), jnp.int32),
          pltpu.VMEM((_B_PER_W,), jnp.int32),
          pltpu.VMEM((_B_PER_W, 2 * _D), jnp.float32),
          pltpu.SemaphoreType.DMA,
      ],
  )
  return fn(ids, y)


_MLP_BLK = 2048
_NBLK = _B // _MLP_BLK


def _mlp_body(yc_ref, yd_ref, cid_ref, did_ref, w1a_ref, w1b_ref, b1_ref,
              w2_ref, b2_ref, w3_ref, b3_ref, w4_ref, b4_ref, out_ref):
  cbit = cid_ref[0, 0, :].reshape(_MLP_BLK, 1)
  dbit = did_ref[0, 0, :].reshape(_MLP_BLK, 1)

  def quarter(y_ref, b):
    u = lax.bitcast_convert_type(y_ref[...], jnp.uint32)
    q = ((b >= _HALF).astype(jnp.int32) + (b >= 2 * _HALF).astype(jnp.int32)
         + (b >= 3 * _HALF).astype(jnp.int32))
    ge2 = q >= 2
    odd = (q & 1) == 1
    uhalf = jnp.where(ge2, u[:, _D:], u[:, :_D])
    ubits = jnp.where(odd, uhalf & jnp.uint32(0xFFFF0000),
                      jnp.left_shift(uhalf, jnp.uint32(16)))
    return lax.bitcast_convert_type(ubits, jnp.float32)

  xc = quarter(yc_ref, cbit)
  xd = quarter(yd_ref, dbit)
  h = jnp.maximum(xc @ w1a_ref[...] + xd @ w1b_ref[...] + b1_ref[...], 0.0)
  h = jnp.maximum(h @ w2_ref[...] + b2_ref[...], 0.0)
  h = jnp.maximum(h @ w3_ref[...] + b3_ref[...], 0.0)
  out_ref[...] = h @ w4_ref[...] + b4_ref[...]


def _mlp(yc, yd, cid3, did3, W1, b1, W2, b2, W3, b3, W4, b4):
  grid = (_NBLK,)
  full = lambda shape: pl.BlockSpec(shape, lambda i: tuple(0 for _ in shape))
  return pl.pallas_call(
      _mlp_body,
      grid=grid,
      in_specs=[
          pl.BlockSpec((_MLP_BLK, 2 * _D), lambda i: (i, 0)),
          pl.BlockSpec((_MLP_BLK, 2 * _D), lambda i: (i, 0)),
          pl.BlockSpec((1, 1, _MLP_BLK), lambda i: (i, 0, 0)),
          pl.BlockSpec((1, 1, _MLP_BLK), lambda i: (i, 0, 0)),
          full((_D, 128)),
          full((_D, 128)),
          full((1, 128)),
          full((128, 64)),
          full((1, 64)),
          full((64, 32)),
          full((1, 32)),
          full((32, 1)),
          full((1, 1)),
      ],
      out_specs=pl.BlockSpec((_MLP_BLK, 1), lambda i: (i, 0)),
      out_shape=jax.ShapeDtypeStruct((_B, 1), jnp.float32),
  )(yc, yd, cid3, did3, W1[:_D], W1[_D:], b1.reshape(1, -1),
    W2, b2.reshape(1, -1), W3, b3.reshape(1, -1), W4, b4.reshape(1, 1))


@jax.jit
def kernel(client_ids, cleaner_ids, client_table, cleaner_table,
           W1, b1, W2, b2, W3, b3, W4, b4):
  cid = client_ids.astype(jnp.int32)
  did = cleaner_ids.astype(jnp.int32)
  yc_tab = _repack(client_table.T)
  yc = _sc_gather(cid, yc_tab)
  yd_tab = _repack(cleaner_table.T)
  yd = _sc_gather(did, yd_tab)
  cid3 = cid.reshape(_NBLK, 1, _MLP_BLK)
  did3 = did.reshape(_NBLK, 1, _MLP_BLK)
  out = _mlp(yc, yd, cid3, did3, W1, b1, W2, b2, W3, b3, W4, b4)
  return out.reshape(_B)


# speedup vs baseline: 31.7430x; 1.1026x over previous
import functools

import jax
import jax.numpy as jnp
from jax import lax
from jax.experimental import pallas as pl
from jax.experimental.pallas import tpu as pltpu
from jax.experimental.pallas import tpu_sc as plsc

_NC = 2
_NS = 16
_NW = _NC * _NS

_B = 16384
_D = 64
_N = 1000000
_B_PER_W = _B // _NW
_L = 16


_RP_CH = 8192
_NSTEP = 31
_HALF = _RP_CH * _NSTEP
_MAXBLK = _N // _RP_CH


def _bf16_bits(x):
  return lax.bitcast_convert_type(
      x.astype(jnp.bfloat16).astype(jnp.float32), jnp.uint32)


def _repack_body(r0_ref, r1_ref, r2_ref, r3_ref, out_ref):
  z01 = (jnp.right_shift(_bf16_bits(r0_ref[...]), jnp.uint32(16))
         | (_bf16_bits(r1_ref[...]) & jnp.uint32(0xFFFF0000)))
  z23 = (jnp.right_shift(_bf16_bits(r2_ref[...]), jnp.uint32(16))
         | (_bf16_bits(r3_ref[...]) & jnp.uint32(0xFFFF0000)))
  packed = jnp.concatenate([z01.T, z23.T], axis=-1)
  out_ref[...] = lax.bitcast_convert_type(packed, jnp.float32)


def _repack(tabT):
  grid = (_NSTEP,)

  def spec(k):
    return pl.BlockSpec(
        (_D, _RP_CH), lambda i: (0, jnp.minimum(i + k * _NSTEP, _MAXBLK)))

  return pl.pallas_call(
      _repack_body,
      grid=grid,
      in_specs=[spec(0), spec(1), spec(2), spec(3)],
      out_specs=pl.BlockSpec((_RP_CH, 2 * _D), lambda i: (i, 0)),
      out_shape=jax.ShapeDtypeStruct((_HALF, 2 * _D), jnp.float32),
  )(tabT, tabT, tabT, tabT)


def _sc_gather_body(ids_hbm, y_hbm, out_hbm, idx, tid, ybuf, sem):
  wid = lax.axis_index("s") * _NC + lax.axis_index("c")
  base = wid * _B_PER_W
  pltpu.sync_copy(ids_hbm.at[pl.ds(base, _B_PER_W)], idx)

  def fold(i, _):
    v = idx[pl.ds(i * _L, _L)]
    v = jnp.where(v < 2 * _HALF, v, v - 2 * _HALF)
    tid[pl.ds(i * _L, _L)] = jnp.where(v < _HALF, v, v - _HALF)
    return 0

  lax.fori_loop(0, _B_PER_W // _L, fold, 0)
  pltpu.async_copy(y_hbm.at[tid], ybuf, sem).wait()
  pltpu.sync_copy(ybuf, out_hbm.at[pl.ds(base, _B_PER_W)])


def _sc_gather(ids, y):
  mesh = plsc.VectorSubcoreMesh(core_axis_name="c", subcore_axis_name="s")
  fn = pl.kernel(
      _sc_gather_body,
      out_type=jax.ShapeDtypeStruct((_B, 2 * _D), jnp.float32),
      mesh=mesh,
      scratch_types=[
          pltpu.VMEM((_B_PER_W,), jnp.int32),
          pltpu.VMEM((_B_PER_W,), jnp.int32),
          pltpu.VMEM((_B_PER_W, 2 * _D), jnp.float32),
          pltpu.SemaphoreType.DMA,
      ],
  )
  return fn(ids, y)


_MLP_BLK = 2048
_NBLK = _B // _MLP_BLK


def _mlp_body(yc_ref, yd_ref, cid_ref, did_ref, w1a_ref, w1b_ref, b1_ref,
              w2_ref, b2_ref, w3_ref, b3_ref, w4_ref, b4_ref, out_ref):
  cbit = cid_ref[0, 0, :].reshape(_MLP_BLK, 1)
  dbit = did_ref[0, 0, :].reshape(_MLP_BLK, 1)

  def quarter(y_ref, b):
    u = lax.bitcast_convert_type(y_ref[...], jnp.uint32)
    q = ((b >= _HALF).astype(jnp.int32) + (b >= 2 * _HALF).astype(jnp.int32)
         + (b >= 3 * _HALF).astype(jnp.int32))
    ge2 = q >= 2
    odd = (q & 1) == 1
    uhalf = jnp.where(ge2, u[:, _D:], u[:, :_D])
    ubits = jnp.where(odd, uhalf & jnp.uint32(0xFFFF0000),
                      jnp.left_shift(uhalf, jnp.uint32(16)))
    return lax.bitcast_convert_type(ubits, jnp.float32)

  xc = quarter(yc_ref, cbit)
  xd = quarter(yd_ref, dbit)
  h = jnp.maximum(xc @ w1a_ref[...] + xd @ w1b_ref[...] + b1_ref[...], 0.0)
  h = jnp.maximum(h @ w2_ref[...] + b2_ref[...], 0.0)
  h = jnp.maximum(h @ w3_ref[...] + b3_ref[...], 0.0)
  out_ref[...] = h @ w4_ref[...] + b4_ref[...]


def _mlp(yc, yd, cid3, did3, W1, b1, W2, b2, W3, b3, W4, b4):
  grid = (_NBLK,)
  full = lambda shape: pl.BlockSpec(shape, lambda i: tuple(0 for _ in shape))
  return pl.pallas_call(
      _mlp_body,
      grid=grid,
      in_specs=[
          pl.BlockSpec((_MLP_BLK, 2 * _D), lambda i: (i, 0)),
          pl.BlockSpec((_MLP_BLK, 2 * _D), lambda i: (i, 0)),
          pl.BlockSpec((1, 1, _MLP_BLK), lambda i: (i, 0, 0)),
          pl.BlockSpec((1, 1, _MLP_BLK), lambda i: (i, 0, 0)),
          full((_D, 128)),
          full((_D, 128)),
          full((1, 128)),
          full((128, 64)),
          full((1, 64)),
          full((64, 32)),
          full((1, 32)),
          full((32, 1)),
          full((1, 1)),
      ],
      out_specs=pl.BlockSpec((_MLP_BLK, 1), lambda i: (i, 0)),
      out_shape=jax.ShapeDtypeStruct((_B, 1), jnp.float32),
  )(yc, yd, cid3, did3, W1[:_D], W1[_D:], b1.reshape(1, -1),
    W2, b2.reshape(1, -1), W3, b3.reshape(1, -1), W4, b4.reshape(1, 1))


@jax.jit
def kernel(client_ids, cleaner_ids, client_table, cleaner_table,
           W1, b1, W2, b2, W3, b3, W4, b4):
  cid = client_ids.astype(jnp.int32)
  did = cleaner_ids.astype(jnp.int32)
  yc_tab = _repack(client_table.T)
  yc = _sc_gather(cid, yc_tab)
  yd_tab = _repack(cleaner_table.T)
  yd = _sc_gather(did, yd_tab)
  cid3 = cid.reshape(_NBLK, 1, _MLP_BLK)
  did3 = did.reshape(_NBLK, 1, _MLP_BLK)
  out = _mlp(yc, yd, cid3, did3, W1, b1, W2, b2, W3, b3, W4, b4)
  return out.reshape(_B)


# speedup vs baseline: 32.5419x; 1.0252x over previous
import functools

import jax
import jax.numpy as jnp
from jax import lax
from jax.experimental import pallas as pl
from jax.experimental.pallas import tpu as pltpu
from jax.experimental.pallas import tpu_sc as plsc

_NC = 2
_NS = 16
_NW = _NC * _NS

_B = 16384
_D = 64
_N = 1000000
_B_PER_W = _B // _NW
_L = 16


_RP_CH = 12288
_NSTEP = 21
_HALF = _RP_CH * _NSTEP
_MAXBLK = _N // _RP_CH


def _bf16_bits(x):
  return lax.bitcast_convert_type(
      x.astype(jnp.bfloat16).astype(jnp.float32), jnp.uint32)


def _repack_body(r0_ref, r1_ref, r2_ref, r3_ref, out_ref):
  z01 = (jnp.right_shift(_bf16_bits(r0_ref[...]), jnp.uint32(16))
         | (_bf16_bits(r1_ref[...]) & jnp.uint32(0xFFFF0000)))
  z23 = (jnp.right_shift(_bf16_bits(r2_ref[...]), jnp.uint32(16))
         | (_bf16_bits(r3_ref[...]) & jnp.uint32(0xFFFF0000)))
  packed = jnp.concatenate([z01.T, z23.T], axis=-1)
  out_ref[...] = lax.bitcast_convert_type(packed, jnp.float32)


def _repack(tabT):
  grid = (_NSTEP,)

  def spec(k):
    return pl.BlockSpec(
        (_D, _RP_CH), lambda i: (0, jnp.minimum(i + k * _NSTEP, _MAXBLK)))

  return pl.pallas_call(
      _repack_body,
      grid=grid,
      in_specs=[spec(0), spec(1), spec(2), spec(3)],
      out_specs=pl.BlockSpec((_RP_CH, 2 * _D), lambda i: (i, 0)),
      out_shape=jax.ShapeDtypeStruct((_HALF, 2 * _D), jnp.float32),
  )(tabT, tabT, tabT, tabT)


def _sc_gather_body(ids_hbm, y_hbm, out_hbm, idx, tid, ybuf, sem):
  wid = lax.axis_index("s") * _NC + lax.axis_index("c")
  base = wid * _B_PER_W
  pltpu.sync_copy(ids_hbm.at[pl.ds(base, _B_PER_W)], idx)

  def fold(i, _):
    v = idx[pl.ds(i * _L, _L)]
    v = jnp.where(v < 2 * _HALF, v, v - 2 * _HALF)
    tid[pl.ds(i * _L, _L)] = jnp.where(v < _HALF, v, v - _HALF)
    return 0

  lax.fori_loop(0, _B_PER_W // _L, fold, 0)
  pltpu.async_copy(y_hbm.at[tid], ybuf, sem).wait()
  pltpu.sync_copy(ybuf, out_hbm.at[pl.ds(base, _B_PER_W)])


def _sc_gather(ids, y):
  mesh = plsc.VectorSubcoreMesh(core_axis_name="c", subcore_axis_name="s")
  fn = pl.kernel(
      _sc_gather_body,
      out_type=jax.ShapeDtypeStruct((_B, 2 * _D), jnp.float32),
      mesh=mesh,
      scratch_types=[
          pltpu.VMEM((_B_PER_W,), jnp.int32),
          pltpu.VMEM((_B_PER_W,), jnp.int32),
          pltpu.VMEM((_B_PER_W, 2 * _D), jnp.float32),
          pltpu.SemaphoreType.DMA,
      ],
  )
  return fn(ids, y)


_MLP_BLK = 2048
_NBLK = _B // _MLP_BLK


def _mlp_body(yc_ref, yd_ref, cid_ref, did_ref, w1a_ref, w1b_ref, b1_ref,
              w2_ref, b2_ref, w3_ref, b3_ref, w4_ref, b4_ref, out_ref):
  cbit = cid_ref[0, 0, :].reshape(_MLP_BLK, 1)
  dbit = did_ref[0, 0, :].reshape(_MLP_BLK, 1)

  def quarter(y_ref, b):
    u = lax.bitcast_convert_type(y_ref[...], jnp.uint32)
    q = ((b >= _HALF).astype(jnp.int32) + (b >= 2 * _HALF).astype(jnp.int32)
         + (b >= 3 * _HALF).astype(jnp.int32))
    ge2 = q >= 2
    odd = (q & 1) == 1
    uhalf = jnp.where(ge2, u[:, _D:], u[:, :_D])
    ubits = jnp.where(odd, uhalf & jnp.uint32(0xFFFF0000),
                      jnp.left_shift(uhalf, jnp.uint32(16)))
    return lax.bitcast_convert_type(ubits, jnp.float32)

  xc = quarter(yc_ref, cbit)
  xd = quarter(yd_ref, dbit)
  h = jnp.maximum(xc @ w1a_ref[...] + xd @ w1b_ref[...] + b1_ref[...], 0.0)
  h = jnp.maximum(h @ w2_ref[...] + b2_ref[...], 0.0)
  h = jnp.maximum(h @ w3_ref[...] + b3_ref[...], 0.0)
  out_ref[...] = h @ w4_ref[...] + b4_ref[...]


def _mlp(yc, yd, cid3, did3, W1, b1, W2, b2, W3, b3, W4, b4):
  grid = (_NBLK,)
  full = lambda shape: pl.BlockSpec(shape, lambda i: tuple(0 for _ in shape))
  return pl.pallas_call(
      _mlp_body,
      grid=grid,
      in_specs=[
          pl.BlockSpec((_MLP_BLK, 2 * _D), lambda i: (i, 0)),
          pl.BlockSpec((_MLP_BLK, 2 * _D), lambda i: (i, 0)),
          pl.BlockSpec((1, 1, _MLP_BLK), lambda i: (i, 0, 0)),
          pl.BlockSpec((1, 1, _MLP_BLK), lambda i: (i, 0, 0)),
          full((_D, 128)),
          full((_D, 128)),
          full((1, 128)),
          full((128, 64)),
          full((1, 64)),
          full((64, 32)),
          full((1, 32)),
          full((32, 1)),
          full((1, 1)),
      ],
      out_specs=pl.BlockSpec((_MLP_BLK, 1), lambda i: (i, 0)),
      out_shape=jax.ShapeDtypeStruct((_B, 1), jnp.float32),
  )(yc, yd, cid3, did3, W1[:_D], W1[_D:], b1.reshape(1, -1),
    W2, b2.reshape(1, -1), W3, b3.reshape(1, -1), W4, b4.reshape(1, 1))


@jax.jit
def kernel(client_ids, cleaner_ids, client_table, cleaner_table,
           W1, b1, W2, b2, W3, b3, W4, b4):
  cid = client_ids.astype(jnp.int32)
  did = cleaner_ids.astype(jnp.int32)
  yc_tab = _repack(client_table.T)
  yc = _sc_gather(cid, yc_tab)
  yd_tab = _repack(cleaner_table.T)
  yd = _sc_gather(did, yd_tab)
  cid3 = cid.reshape(_NBLK, 1, _MLP_BLK)
  did3 = did.reshape(_NBLK, 1, _MLP_BLK)
  out = _mlp(yc, yd, cid3, did3, W1, b1, W2, b2, W3, b3, W4, b4)
  return out.reshape(_B)
